# baseline (speedup 1.0000x reference)
import functools

import jax
import jax.numpy as jnp
from jax import lax
from jax.experimental import pallas as pl
from jax.experimental.pallas import tpu as pltpu
from jax.experimental.pallas import tpu_sc as plsc

R = 16384
C = 2048
BR1 = 1024
BR = 1024
NC = 2
NS = 16
NW = NC * NS
L = 16
RSC = 4096
RA = R - RSC
RPW = RSC // NW
G = 4
NBUF = 4

_F32 = jnp.float32
_INF = float("inf")


def _minmax_body(x_ref, lab_ref, o_ref, acc_ref):
    i = pl.program_id(0)
    col = lax.broadcasted_iota(jnp.int32, (8, 128), 1)

    @pl.when(i == 0)
    def _():
        acc_ref[...] = jnp.full((8, 128), _INF, _F32)

    x = x_ref[...]
    lab = lab_ref[...]
    mn = jnp.min(x, axis=1, keepdims=True)
    mx = jnp.max(x, axis=1, keepdims=True)
    step = jnp.full((8, 128), _INF, _F32)
    for lev in range(3):
        m = lab == lev
        bmin = jnp.min(jnp.where(m, mn, _INF))
        bneg = jnp.min(jnp.where(m, -mx, _INF))
        step = jnp.where(col == lev, bmin, step)
        step = jnp.where(col == 3 + lev, bneg, step)
    acc_ref[...] = jnp.minimum(acc_ref[...], step)
    o_ref[...] = acc_ref[...]


def _level_minmax(tensor, labels2d):
    return pl.pallas_call(
        _minmax_body,
        grid=(RA // BR1,),
        in_specs=[
            pl.BlockSpec((BR1, C), lambda i: (i, 0)),
            pl.BlockSpec((BR1, 1), lambda i: (i, 0)),
        ],
        out_specs=pl.BlockSpec((8, 128), lambda i: (0, 0)),
        out_shape=jax.ShapeDtypeStruct((8, 128), _F32),
        scratch_shapes=[pltpu.VMEM((8, 128), _F32)],
    )(tensor, labels2d)


def _sc_minmax_body(
    tensor_hbm, labels_hbm,
    sumb_hbm,
    buf, lab_v, st_v, sem0, sem1, sem2, sem3,
):
    cid = lax.axis_index("c")
    sid = lax.axis_index("s")
    wid = sid * NC + cid
    r0 = RA + wid * RPW

    pltpu.sync_copy(labels_hbm.at[pl.ds(wid * RPW + RA, RPW)], lab_v)

    sems = (sem0, sem1, sem2, sem3)
    for b in range(NBUF):
        pltpu.async_copy(tensor_hbm.at[pl.ds(r0 + b * G, G)],
                         buf.at[b], sems[b])

    def process_row(r, rr, accs, b):
        rmins = [buf[b, rr, pl.ds(k * L, L)] for k in range(4)]
        rmaxs = list(rmins)
        for c in range(4, C // L):
            k = c % 4
            x = buf[b, rr, pl.ds(c * L, L)]
            rmins[k] = jnp.minimum(rmins[k], x)
            rmaxs[k] = jnp.maximum(rmaxs[k], x)
        rmin = jnp.minimum(jnp.minimum(rmins[0], rmins[1]),
                           jnp.minimum(rmins[2], rmins[3]))
        rmax = jnp.maximum(jnp.maximum(rmaxs[0], rmaxs[1]),
                           jnp.maximum(rmaxs[2], rmaxs[3]))
        labv = plsc.load_gather(lab_v, [jnp.full((L,), r, jnp.int32)])
        out = []
        for lev in range(3):
            m = labv == lev
            out.append(jnp.minimum(accs[2 * lev], jnp.where(m, rmin, _INF)))
            out.append(jnp.minimum(accs[2 * lev + 1],
                                   jnp.where(m, -rmax, _INF)))
        return out

    def ring_body(p, accs):
        accs = list(accs)
        base = NBUF * G * p
        for b in range(NBUF):
            gbase = base + b * G
            pltpu.make_async_copy(
                tensor_hbm.at[pl.ds(r0, G)], buf.at[b], sems[b]).wait()
            for rr in range(G):
                accs = process_row(gbase + rr, rr, accs, b)

            @pl.when(gbase + (NBUF + 1) * G <= RPW)
            def _():
                pltpu.async_copy(
                    tensor_hbm.at[pl.ds(r0 + gbase + NBUF * G, G)],
                    buf.at[b], sems[b])
        return tuple(accs)

    init = tuple(jnp.full((L,), _INF, _F32) for _ in range(6))
    accs = lax.fori_loop(0, RPW // (NBUF * G), ring_body, init, unroll=False)

    for a in range(6):
        st_v[...] = accs[a]
        pltpu.sync_copy(st_v, sumb_hbm.at[a * NW + wid])


@functools.cache
def _sc_minmax_call():
    return functools.partial(
        pl.kernel,
        out_type=jax.ShapeDtypeStruct((6 * NW, L), _F32),
        mesh=plsc.VectorSubcoreMesh(
            core_axis_name="c", subcore_axis_name="s",
            num_cores=NC, num_subcores=NS),
        scratch_types=[
            pltpu.VMEM((NBUF, G, C), _F32),
            pltpu.VMEM((RPW,), jnp.int32),
            pltpu.VMEM((L,), _F32),
            pltpu.SemaphoreType.DMA,
            pltpu.SemaphoreType.DMA,
            pltpu.SemaphoreType.DMA,
            pltpu.SemaphoreType.DMA,
        ],
        compiler_params=pltpu.CompilerParams(needs_layout_passes=False),
    )(_sc_minmax_body)


def _quant_body(x_ref, lab_ref, suma_ref, sumb_ref, o_ref, prm_ref):
    i = pl.program_id(0)

    @pl.when(i == 0)
    def _():
        sb = sumb_ref[...]
        rown = lax.broadcasted_iota(jnp.int32, (6 * NW, L), 0) // NW
        for lev in range(3):
            mnb = jnp.min(jnp.where(rown == 2 * lev, sb, _INF))
            ngb = jnp.min(jnp.where(rown == 2 * lev + 1, sb, _INF))
            mnv = jnp.minimum(suma_ref[0, lev], mnb)
            ngv = jnp.minimum(suma_ref[0, 3 + lev], ngb)
            mxv = -ngv
            qmax = float(2 ** (2 ** (lev + 1)) - 1)
            deg = jnp.logical_not(mxv > mnv)
            safe = jnp.where(deg, 1.0, mxv - mnv)
            scale = jnp.where(deg, 1.0, safe / qmax)
            zp = jnp.where(deg, 0.0, -mnv / scale)
            prm_ref[lev] = scale
            prm_ref[3 + lev] = zp

    x = x_ref[...]
    lab = lab_ref[...]
    s = jnp.where(lab == 0, prm_ref[0],
                  jnp.where(lab == 1, prm_ref[1], prm_ref[2]))
    z = jnp.where(lab == 0, prm_ref[3],
                  jnp.where(lab == 1, prm_ref[4], prm_ref[5]))
    qm = jnp.where(lab == 0, 3.0, jnp.where(lab == 1, 15.0, 255.0))
    y = x / s + z
    y = jnp.clip(y, 0.0, qm)
    q = jnp.round(y)
    o_ref[...] = (q - z) * s


def _quantize(tensor, labels2d, suma, sumb):
    return pl.pallas_call(
        _quant_body,
        grid=(R // BR,),
        in_specs=[
            pl.BlockSpec((BR, C), lambda i: (i, 0)),
            pl.BlockSpec((BR, 1), lambda i: (i, 0)),
            pl.BlockSpec(memory_space=pltpu.SMEM),
            pl.BlockSpec((6 * NW, L), lambda i: (0, 0)),
        ],
        out_specs=pl.BlockSpec((BR, C), lambda i: (i, 0)),
        out_shape=jax.ShapeDtypeStruct((R, C), _F32),
        scratch_shapes=[pltpu.SMEM((8,), _F32)],
    )(tensor, labels2d, suma, sumb)


def kernel(tensor, precision_labels):
    labels2d = precision_labels.reshape(R, 1)
    suma = _level_minmax(tensor, labels2d)
    sumb = _sc_minmax_call()(tensor, precision_labels)
    return _quantize(tensor, labels2d, suma, sumb)


# speedup vs baseline: 1.1744x; 1.1744x over previous
import functools

import jax
import jax.numpy as jnp
from jax import lax
from jax.experimental import pallas as pl
from jax.experimental.pallas import tpu as pltpu
from jax.experimental.pallas import tpu_sc as plsc

R = 16384
C = 2048
BR1 = 1024
BR = 1024
NC = 2
NS = 16
NW = NC * NS
L = 16
RSC = 2048
RA = R - RSC
RPW = RSC // NW
G = 4
NBUF = 4

_F32 = jnp.float32
_INF = float("inf")


def _minmax_body(x_ref, lab_ref, o_ref, acc_ref):
    i = pl.program_id(0)
    col = lax.broadcasted_iota(jnp.int32, (8, 128), 1)

    @pl.when(i == 0)
    def _():
        acc_ref[...] = jnp.full((8, 128), _INF, _F32)

    x = x_ref[...]
    lab = lab_ref[...]
    mn = jnp.min(x, axis=1, keepdims=True)
    mx = jnp.max(x, axis=1, keepdims=True)
    step = jnp.full((8, 128), _INF, _F32)
    for lev in range(3):
        m = lab == lev
        bmin = jnp.min(jnp.where(m, mn, _INF))
        bneg = jnp.min(jnp.where(m, -mx, _INF))
        step = jnp.where(col == lev, bmin, step)
        step = jnp.where(col == 3 + lev, bneg, step)
    acc_ref[...] = jnp.minimum(acc_ref[...], step)
    o_ref[...] = acc_ref[...]


def _level_minmax(tensor, labels2d):
    return pl.pallas_call(
        _minmax_body,
        grid=(RA // BR1,),
        in_specs=[
            pl.BlockSpec((BR1, C), lambda i: (i, 0)),
            pl.BlockSpec((BR1, 1), lambda i: (i, 0)),
        ],
        out_specs=pl.BlockSpec((8, 128), lambda i: (0, 0)),
        out_shape=jax.ShapeDtypeStruct((8, 128), _F32),
        scratch_shapes=[pltpu.VMEM((8, 128), _F32)],
    )(tensor, labels2d)


def _sc_minmax_body(
    tensor_hbm, labels_hbm,
    sumb_hbm,
    buf, lab_v, st_v, sem0, sem1, sem2, sem3,
):
    cid = lax.axis_index("c")
    sid = lax.axis_index("s")
    wid = sid * NC + cid
    r0 = RA + wid * RPW

    pltpu.sync_copy(labels_hbm.at[pl.ds(wid * RPW + RA, RPW)], lab_v)

    sems = (sem0, sem1, sem2, sem3)
    for b in range(NBUF):
        pltpu.async_copy(tensor_hbm.at[pl.ds(r0 + b * G, G)],
                         buf.at[b], sems[b])

    def process_row(r, rr, accs, b):
        NCH = 8
        rmins = [buf[b, rr, pl.ds(k * L, L)] for k in range(NCH)]
        rmaxs = list(rmins)
        for c in range(NCH, C // L):
            k = c % NCH
            x = buf[b, rr, pl.ds(c * L, L)]
            rmins[k] = jnp.minimum(rmins[k], x)
            rmaxs[k] = jnp.maximum(rmaxs[k], x)
        while len(rmins) > 1:
            rmins = [jnp.minimum(a, b2)
                     for a, b2 in zip(rmins[::2], rmins[1::2])]
            rmaxs = [jnp.maximum(a, b2)
                     for a, b2 in zip(rmaxs[::2], rmaxs[1::2])]
        rmin, rmax = rmins[0], rmaxs[0]
        labv = plsc.load_gather(lab_v, [jnp.full((L,), r, jnp.int32)])
        out = []
        for lev in range(3):
            m = labv == lev
            out.append(jnp.minimum(accs[2 * lev], jnp.where(m, rmin, _INF)))
            out.append(jnp.minimum(accs[2 * lev + 1],
                                   jnp.where(m, -rmax, _INF)))
        return out

    def ring_body(p, accs):
        accs = list(accs)
        base = NBUF * G * p
        for b in range(NBUF):
            gbase = base + b * G
            pltpu.make_async_copy(
                tensor_hbm.at[pl.ds(r0, G)], buf.at[b], sems[b]).wait()
            for rr in range(G):
                accs = process_row(gbase + rr, rr, accs, b)

            @pl.when(gbase + (NBUF + 1) * G <= RPW)
            def _():
                pltpu.async_copy(
                    tensor_hbm.at[pl.ds(r0 + gbase + NBUF * G, G)],
                    buf.at[b], sems[b])
        return tuple(accs)

    init = tuple(jnp.full((L,), _INF, _F32) for _ in range(6))
    accs = lax.fori_loop(0, RPW // (NBUF * G), ring_body, init, unroll=False)

    for a in range(6):
        st_v[...] = accs[a]
        pltpu.sync_copy(st_v, sumb_hbm.at[a * NW + wid])


@functools.cache
def _sc_minmax_call():
    return functools.partial(
        pl.kernel,
        out_type=jax.ShapeDtypeStruct((6 * NW, L), _F32),
        mesh=plsc.VectorSubcoreMesh(
            core_axis_name="c", subcore_axis_name="s",
            num_cores=NC, num_subcores=NS),
        scratch_types=[
            pltpu.VMEM((NBUF, G, C), _F32),
            pltpu.VMEM((RPW,), jnp.int32),
            pltpu.VMEM((L,), _F32),
            pltpu.SemaphoreType.DMA,
            pltpu.SemaphoreType.DMA,
            pltpu.SemaphoreType.DMA,
            pltpu.SemaphoreType.DMA,
        ],
        compiler_params=pltpu.CompilerParams(needs_layout_passes=False),
    )(_sc_minmax_body)


def _quant_body(x_ref, lab_ref, suma_ref, sumb_ref, o_ref, prm_ref):
    i = pl.program_id(0)

    @pl.when(i == 0)
    def _():
        sb = sumb_ref[...]
        rown = lax.broadcasted_iota(jnp.int32, (6 * NW, L), 0) // NW
        for lev in range(3):
            mnb = jnp.min(jnp.where(rown == 2 * lev, sb, _INF))
            ngb = jnp.min(jnp.where(rown == 2 * lev + 1, sb, _INF))
            mnv = jnp.minimum(suma_ref[0, lev], mnb)
            ngv = jnp.minimum(suma_ref[0, 3 + lev], ngb)
            mxv = -ngv
            qmax = float(2 ** (2 ** (lev + 1)) - 1)
            deg = jnp.logical_not(mxv > mnv)
            safe = jnp.where(deg, 1.0, mxv - mnv)
            scale = jnp.where(deg, 1.0, safe / qmax)
            zp = jnp.where(deg, 0.0, -mnv / scale)
            prm_ref[lev] = scale
            prm_ref[3 + lev] = zp

    x = x_ref[...]
    lab = lab_ref[...]
    s = jnp.where(lab == 0, prm_ref[0],
                  jnp.where(lab == 1, prm_ref[1], prm_ref[2]))
    z = jnp.where(lab == 0, prm_ref[3],
                  jnp.where(lab == 1, prm_ref[4], prm_ref[5]))
    qm = jnp.where(lab == 0, 3.0, jnp.where(lab == 1, 15.0, 255.0))
    y = x / s + z
    y = jnp.clip(y, 0.0, qm)
    q = jnp.round(y)
    o_ref[...] = (q - z) * s


def _quantize(tensor, labels2d, suma, sumb):
    return pl.pallas_call(
        _quant_body,
        grid=(R // BR,),
        in_specs=[
            pl.BlockSpec((BR, C), lambda i: (i, 0)),
            pl.BlockSpec((BR, 1), lambda i: (i, 0)),
            pl.BlockSpec(memory_space=pltpu.SMEM),
            pl.BlockSpec((6 * NW, L), lambda i: (0, 0)),
        ],
        out_specs=pl.BlockSpec((BR, C), lambda i: (i, 0)),
        out_shape=jax.ShapeDtypeStruct((R, C), _F32),
        scratch_shapes=[pltpu.SMEM((8,), _F32)],
    )(tensor, labels2d, suma, sumb)


def kernel(tensor, precision_labels):
    labels2d = precision_labels.reshape(R, 1)
    suma = _level_minmax(tensor, labels2d)
    sumb = _sc_minmax_call()(tensor, precision_labels)
    return _quantize(tensor, labels2d, suma, sumb)


# speedup vs baseline: 1.1773x; 1.0025x over previous
import functools

import jax
import jax.numpy as jnp
from jax import lax
from jax.experimental import pallas as pl
from jax.experimental.pallas import tpu as pltpu
from jax.experimental.pallas import tpu_sc as plsc

R = 16384
C = 2048
BR1 = 1024
BR = 1024
NC = 2
NS = 16
NW = NC * NS
L = 16
RSC = 2048
RA = R - RSC
RPW = RSC // NW
G = 2
NBUF = 4

_F32 = jnp.float32
_INF = float("inf")


def _minmax_body(x_ref, lab_ref, o_ref, acc_ref):
    i = pl.program_id(0)
    col = lax.broadcasted_iota(jnp.int32, (8, 128), 1)

    @pl.when(i == 0)
    def _():
        acc_ref[...] = jnp.full((8, 128), _INF, _F32)

    x = x_ref[...]
    lab = lab_ref[...]
    mn = jnp.min(x, axis=1, keepdims=True)
    mx = jnp.max(x, axis=1, keepdims=True)
    step = jnp.full((8, 128), _INF, _F32)
    for lev in range(3):
        m = lab == lev
        bmin = jnp.min(jnp.where(m, mn, _INF))
        bneg = jnp.min(jnp.where(m, -mx, _INF))
        step = jnp.where(col == lev, bmin, step)
        step = jnp.where(col == 3 + lev, bneg, step)
    acc_ref[...] = jnp.minimum(acc_ref[...], step)
    o_ref[...] = acc_ref[...]


def _level_minmax(tensor, labels2d):
    return pl.pallas_call(
        _minmax_body,
        grid=(RA // BR1,),
        in_specs=[
            pl.BlockSpec((BR1, C), lambda i: (i, 0)),
            pl.BlockSpec((BR1, 1), lambda i: (i, 0)),
        ],
        out_specs=pl.BlockSpec((8, 128), lambda i: (0, 0)),
        out_shape=jax.ShapeDtypeStruct((8, 128), _F32),
        scratch_shapes=[pltpu.VMEM((8, 128), _F32)],
    )(tensor, labels2d)


def _sc_minmax_body(
    tensor_hbm, labels_hbm,
    sumb_hbm,
    buf, lab_v, st_v, sem0, sem1, sem2, sem3,
):
    cid = lax.axis_index("c")
    sid = lax.axis_index("s")
    wid = sid * NC + cid
    r0 = RA + wid * RPW

    pltpu.sync_copy(labels_hbm.at[pl.ds(wid * RPW + RA, RPW)], lab_v)

    sems = (sem0, sem1, sem2, sem3)
    for b in range(NBUF):
        pltpu.async_copy(tensor_hbm.at[pl.ds(r0 + b * G, G)],
                         buf.at[b], sems[b])

    def process_row(r, rr, accs, b):
        NCH = 4
        rmins = [buf[b, rr, pl.ds(k * L, L)] for k in range(NCH)]
        rmaxs = list(rmins)
        for c in range(NCH, C // L):
            k = c % NCH
            x = buf[b, rr, pl.ds(c * L, L)]
            rmins[k] = jnp.minimum(rmins[k], x)
            rmaxs[k] = jnp.maximum(rmaxs[k], x)
        while len(rmins) > 1:
            rmins = [jnp.minimum(a, b2)
                     for a, b2 in zip(rmins[::2], rmins[1::2])]
            rmaxs = [jnp.maximum(a, b2)
                     for a, b2 in zip(rmaxs[::2], rmaxs[1::2])]
        rmin, rmax = rmins[0], rmaxs[0]
        labv = plsc.load_gather(lab_v, [jnp.full((L,), r, jnp.int32)])
        out = []
        for lev in range(3):
            m = labv == lev
            out.append(jnp.minimum(accs[2 * lev], jnp.where(m, rmin, _INF)))
            out.append(jnp.minimum(accs[2 * lev + 1],
                                   jnp.where(m, -rmax, _INF)))
        return out

    def ring_body(p, accs):
        accs = list(accs)
        base = NBUF * G * p
        for b in range(NBUF):
            gbase = base + b * G
            pltpu.make_async_copy(
                tensor_hbm.at[pl.ds(r0, G)], buf.at[b], sems[b]).wait()
            for rr in range(G):
                accs = process_row(gbase + rr, rr, accs, b)

            @pl.when(gbase + (NBUF + 1) * G <= RPW)
            def _():
                pltpu.async_copy(
                    tensor_hbm.at[pl.ds(r0 + gbase + NBUF * G, G)],
                    buf.at[b], sems[b])
        return tuple(accs)

    init = tuple(jnp.full((L,), _INF, _F32) for _ in range(6))
    accs = lax.fori_loop(0, RPW // (NBUF * G), ring_body, init, unroll=False)

    for a in range(6):
        st_v[...] = accs[a]
        pltpu.sync_copy(st_v, sumb_hbm.at[a * NW + wid])


@functools.cache
def _sc_minmax_call():
    return functools.partial(
        pl.kernel,
        out_type=jax.ShapeDtypeStruct((6 * NW, L), _F32),
        mesh=plsc.VectorSubcoreMesh(
            core_axis_name="c", subcore_axis_name="s",
            num_cores=NC, num_subcores=NS),
        scratch_types=[
            pltpu.VMEM((NBUF, G, C), _F32),
            pltpu.VMEM((RPW,), jnp.int32),
            pltpu.VMEM((L,), _F32),
            pltpu.SemaphoreType.DMA,
            pltpu.SemaphoreType.DMA,
            pltpu.SemaphoreType.DMA,
            pltpu.SemaphoreType.DMA,
        ],
        compiler_params=pltpu.CompilerParams(needs_layout_passes=False),
    )(_sc_minmax_body)


def _quant_body(x_ref, lab_ref, suma_ref, sumb_ref, o_ref, prm_ref):
    i = pl.program_id(0)

    @pl.when(i == 0)
    def _():
        sb = sumb_ref[...]
        rown = lax.broadcasted_iota(jnp.int32, (6 * NW, L), 0) // NW
        for lev in range(3):
            mnb = jnp.min(jnp.where(rown == 2 * lev, sb, _INF))
            ngb = jnp.min(jnp.where(rown == 2 * lev + 1, sb, _INF))
            mnv = jnp.minimum(suma_ref[0, lev], mnb)
            ngv = jnp.minimum(suma_ref[0, 3 + lev], ngb)
            mxv = -ngv
            qmax = float(2 ** (2 ** (lev + 1)) - 1)
            deg = jnp.logical_not(mxv > mnv)
            safe = jnp.where(deg, 1.0, mxv - mnv)
            scale = jnp.where(deg, 1.0, safe / qmax)
            zp = jnp.where(deg, 0.0, -mnv / scale)
            prm_ref[lev] = scale
            prm_ref[3 + lev] = zp

    x = x_ref[...]
    lab = lab_ref[...]
    s = jnp.where(lab == 0, prm_ref[0],
                  jnp.where(lab == 1, prm_ref[1], prm_ref[2]))
    z = jnp.where(lab == 0, prm_ref[3],
                  jnp.where(lab == 1, prm_ref[4], prm_ref[5]))
    qm = jnp.where(lab == 0, 3.0, jnp.where(lab == 1, 15.0, 255.0))
    y = x / s + z
    y = jnp.clip(y, 0.0, qm)
    q = jnp.round(y)
    o_ref[...] = (q - z) * s


def _quantize(tensor, labels2d, suma, sumb):
    return pl.pallas_call(
        _quant_body,
        grid=(R // BR,),
        in_specs=[
            pl.BlockSpec((BR, C), lambda i: (i, 0)),
            pl.BlockSpec((BR, 1), lambda i: (i, 0)),
            pl.BlockSpec(memory_space=pltpu.SMEM),
            pl.BlockSpec((6 * NW, L), lambda i: (0, 0)),
        ],
        out_specs=pl.BlockSpec((BR, C), lambda i: (i, 0)),
        out_shape=jax.ShapeDtypeStruct((R, C), _F32),
        scratch_shapes=[pltpu.SMEM((8,), _F32)],
    )(tensor, labels2d, suma, sumb)


def kernel(tensor, precision_labels):
    labels2d = precision_labels.reshape(R, 1)
    suma = _level_minmax(tensor, labels2d)
    sumb = _sc_minmax_call()(tensor, precision_labels)
    return _quantize(tensor, labels2d, suma, sumb)


# speedup vs baseline: 1.1808x; 1.0030x over previous
import functools

import jax
import jax.numpy as jnp
from jax import lax
from jax.experimental import pallas as pl
from jax.experimental.pallas import tpu as pltpu
from jax.experimental.pallas import tpu_sc as plsc

R = 16384
C = 2048
BR1 = 1024
BR = 1024
NC = 2
NS = 16
NW = NC * NS
L = 16
RSC = 2048
RA = R - RSC
RPW = RSC // NW
G = 2
NBUF = 4

_F32 = jnp.float32
_INF = float("inf")


def _minmax_body(x_ref, lab_ref, o_ref, acc_ref):
    i = pl.program_id(0)
    col = lax.broadcasted_iota(jnp.int32, (8, 128), 1)

    @pl.when(i == 0)
    def _():
        acc_ref[...] = jnp.full((8, 128), _INF, _F32)

    x = x_ref[...]
    lab = lab_ref[...]
    mn = jnp.min(x, axis=1, keepdims=True)
    mx = jnp.max(x, axis=1, keepdims=True)
    step = jnp.full((8, 128), _INF, _F32)
    for lev in range(3):
        m = lab == lev
        bmin = jnp.min(jnp.where(m, mn, _INF))
        bneg = jnp.min(jnp.where(m, -mx, _INF))
        step = jnp.where(col == lev, bmin, step)
        step = jnp.where(col == 3 + lev, bneg, step)
    acc_ref[...] = jnp.minimum(acc_ref[...], step)
    o_ref[...] = acc_ref[...]


def _level_minmax(tensor, labels2d):
    return pl.pallas_call(
        _minmax_body,
        grid=(RA // BR1,),
        in_specs=[
            pl.BlockSpec((BR1, C), lambda i: (i, 0)),
            pl.BlockSpec((BR1, 1), lambda i: (i, 0)),
        ],
        out_specs=pl.BlockSpec((8, 128), lambda i: (0, 0)),
        out_shape=jax.ShapeDtypeStruct((8, 128), _F32),
        scratch_shapes=[pltpu.VMEM((8, 128), _F32)],
    )(tensor, labels2d)


def _sc_minmax_body(
    tensor_hbm, labels_hbm,
    sumb_hbm,
    buf, lab_v, st_v, sem0, sem1, sem2, sem3,
):
    cid = lax.axis_index("c")
    sid = lax.axis_index("s")
    wid = sid * NC + cid
    r0 = RA + wid * RPW

    pltpu.sync_copy(labels_hbm.at[pl.ds(wid * RPW + RA, RPW), pl.ds(0, 1)],
                    lab_v)

    sems = (sem0, sem1, sem2, sem3)
    for b in range(NBUF):
        pltpu.async_copy(tensor_hbm.at[pl.ds(r0 + b * G, G)],
                         buf.at[b], sems[b])

    def process_row(r, rr, accs, b):
        NCH = 4
        rmins = [buf[b, rr, pl.ds(k * L, L)] for k in range(NCH)]
        rmaxs = list(rmins)
        for c in range(NCH, C // L):
            k = c % NCH
            x = buf[b, rr, pl.ds(c * L, L)]
            rmins[k] = jnp.minimum(rmins[k], x)
            rmaxs[k] = jnp.maximum(rmaxs[k], x)
        while len(rmins) > 1:
            rmins = [jnp.minimum(a, b2)
                     for a, b2 in zip(rmins[::2], rmins[1::2])]
            rmaxs = [jnp.maximum(a, b2)
                     for a, b2 in zip(rmaxs[::2], rmaxs[1::2])]
        rmin, rmax = rmins[0], rmaxs[0]
        labv = plsc.load_gather(
            lab_v, [jnp.full((L,), r, jnp.int32),
                    jnp.zeros((L,), jnp.int32)])
        out = []
        for lev in range(3):
            m = labv == lev
            out.append(jnp.minimum(accs[2 * lev], jnp.where(m, rmin, _INF)))
            out.append(jnp.minimum(accs[2 * lev + 1],
                                   jnp.where(m, -rmax, _INF)))
        return out

    def ring_body(p, accs):
        accs = list(accs)
        base = NBUF * G * p
        for b in range(NBUF):
            gbase = base + b * G
            pltpu.make_async_copy(
                tensor_hbm.at[pl.ds(r0, G)], buf.at[b], sems[b]).wait()
            for rr in range(G):
                accs = process_row(gbase + rr, rr, accs, b)

            @pl.when(gbase + (NBUF + 1) * G <= RPW)
            def _():
                pltpu.async_copy(
                    tensor_hbm.at[pl.ds(r0 + gbase + NBUF * G, G)],
                    buf.at[b], sems[b])
        return tuple(accs)

    init = tuple(jnp.full((L,), _INF, _F32) for _ in range(6))
    accs = lax.fori_loop(0, RPW // (NBUF * G), ring_body, init, unroll=False)

    for a in range(6):
        st_v[...] = accs[a]
        pltpu.sync_copy(st_v, sumb_hbm.at[a * NW + wid])


@functools.cache
def _sc_minmax_call():
    return functools.partial(
        pl.kernel,
        out_type=jax.ShapeDtypeStruct((6 * NW, L), _F32),
        mesh=plsc.VectorSubcoreMesh(
            core_axis_name="c", subcore_axis_name="s",
            num_cores=NC, num_subcores=NS),
        scratch_types=[
            pltpu.VMEM((NBUF, G, C), _F32),
            pltpu.VMEM((RPW, 1), jnp.int32),
            pltpu.VMEM((L,), _F32),
            pltpu.SemaphoreType.DMA,
            pltpu.SemaphoreType.DMA,
            pltpu.SemaphoreType.DMA,
            pltpu.SemaphoreType.DMA,
        ],
        compiler_params=pltpu.CompilerParams(needs_layout_passes=False),
    )(_sc_minmax_body)


def _quant_body(x_ref, lab_ref, suma_ref, sumb_ref, o_ref, prm_ref):
    i = pl.program_id(0)

    @pl.when(i == 0)
    def _():
        sb = sumb_ref[...]
        rown = lax.broadcasted_iota(jnp.int32, (6 * NW, L), 0) // NW
        for lev in range(3):
            mnb = jnp.min(jnp.where(rown == 2 * lev, sb, _INF))
            ngb = jnp.min(jnp.where(rown == 2 * lev + 1, sb, _INF))
            mnv = jnp.minimum(suma_ref[0, lev], mnb)
            ngv = jnp.minimum(suma_ref[0, 3 + lev], ngb)
            mxv = -ngv
            qmax = float(2 ** (2 ** (lev + 1)) - 1)
            deg = jnp.logical_not(mxv > mnv)
            safe = jnp.where(deg, 1.0, mxv - mnv)
            scale = jnp.where(deg, 1.0, safe / qmax)
            zp = jnp.where(deg, 0.0, -mnv / scale)
            prm_ref[lev] = scale
            prm_ref[3 + lev] = zp

    x = x_ref[...]
    lab = lab_ref[...]
    s = jnp.where(lab == 0, prm_ref[0],
                  jnp.where(lab == 1, prm_ref[1], prm_ref[2]))
    z = jnp.where(lab == 0, prm_ref[3],
                  jnp.where(lab == 1, prm_ref[4], prm_ref[5]))
    qm = jnp.where(lab == 0, 3.0, jnp.where(lab == 1, 15.0, 255.0))
    y = x / s + z
    y = jnp.clip(y, 0.0, qm)
    q = jnp.round(y)
    o_ref[...] = (q - z) * s


def _quantize(tensor, labels2d, suma, sumb):
    return pl.pallas_call(
        _quant_body,
        grid=(R // BR,),
        in_specs=[
            pl.BlockSpec((BR, C), lambda i: (i, 0)),
            pl.BlockSpec((BR, 1), lambda i: (i, 0)),
            pl.BlockSpec(memory_space=pltpu.SMEM),
            pl.BlockSpec((6 * NW, L), lambda i: (0, 0)),
        ],
        out_specs=pl.BlockSpec((BR, C), lambda i: (i, 0)),
        out_shape=jax.ShapeDtypeStruct((R, C), _F32),
        scratch_shapes=[pltpu.SMEM((8,), _F32)],
    )(tensor, labels2d, suma, sumb)


def kernel(tensor, precision_labels):
    labels2d = precision_labels.reshape(R, 1)
    suma = _level_minmax(tensor, labels2d)
    sumb = _sc_minmax_call()(tensor, labels2d)
    return _quantize(tensor, labels2d, suma, sumb)
